# baseline (speedup 1.0000x reference)
import functools

import jax
import jax.numpy as jnp
from jax import lax
from jax.experimental import pallas as pl
from jax.experimental.pallas import tpu as pltpu
from jax.experimental.pallas import tpu_sc as plsc

E = 8
TOP_K = 2
DIM = 1024
HIDDEN = 2816
S = 2048
R = S * TOP_K
BLK = 256
R_PAD = R + E * BLK
W_MAX = R_PAD // BLK


def _shift_down(x, k):
    z = jnp.zeros((k,) + x.shape[1:], x.dtype)
    return jnp.concatenate([z, x[:-k]], axis=0)


def _shift_right(x, k):
    z = jnp.zeros(x.shape[:1] + (k,), x.dtype)
    return jnp.concatenate([z, x[:, :-k]], axis=1)


def _gate_body(x_ref, wg_ref, pos0_ref, pos1_ref, w0_ref, w1_ref,
               we_ref, used_ref):
    x = x_ref[...]
    wg = wg_ref[...]
    s = lax.dot_general(x, wg, (((1,), (1,)), ((), ())),
                        preferred_element_type=jnp.float32)
    col = lax.broadcasted_iota(jnp.int32, s.shape, 1)
    m1 = jnp.max(s, axis=1, keepdims=True)
    i1 = jnp.min(jnp.where(s == m1, col, E), axis=1, keepdims=True)
    oh0 = (col == i1)
    s2 = jnp.where(oh0, -jnp.inf, s)
    m2 = jnp.max(s2, axis=1, keepdims=True)
    i2 = jnp.min(jnp.where(s2 == m2, col, E), axis=1, keepdims=True)
    oh1 = (col == i2)
    e2 = jnp.exp(m2 - m1)
    w0_ref[...] = jnp.broadcast_to(1.0 / (1.0 + e2), (S, 16))
    w1_ref[...] = jnp.broadcast_to(e2 / (1.0 + e2), (S, 16))

    a = oh0.astype(jnp.int32) + oh1.astype(jnp.int32)
    incl = a
    k = 1
    while k < S:
        incl = incl + _shift_down(incl, k)
        k *= 2
    cumbef = incl - a
    cnt = incl[S - 1:S, :]
    blocks_e = (cnt + (BLK - 1)) // BLK
    padc = blocks_e * BLK
    ip = padc
    k = 1
    while k < E:
        ip = ip + _shift_right(ip, k)
        k *= 2
    starts_pad = ip - padc
    base = starts_pad + cumbef
    pos0_ref[...] = jnp.sum(jnp.where(oh0, base, 0), axis=1, keepdims=True)
    pos1_ref[...] = jnp.sum(jnp.where(oh1, base, 0), axis=1, keepdims=True)
    used_ref[...] = jnp.sum(blocks_e, axis=1, keepdims=True)
    start_blk = starts_pad // BLK
    blk = lax.broadcasted_iota(jnp.int32, (W_MAX, E), 0)
    ge = (blk >= jnp.broadcast_to(start_blk, (W_MAX, E))).astype(jnp.int32)
    we_ref[...] = jnp.sum(ge, axis=1, keepdims=True) - 1


def _gate(xf, Wg):
    return pl.pallas_call(
        _gate_body,
        out_shape=(
            jax.ShapeDtypeStruct((S, 1), jnp.int32),
            jax.ShapeDtypeStruct((S, 1), jnp.int32),
            jax.ShapeDtypeStruct((S, 16), jnp.float32),
            jax.ShapeDtypeStruct((S, 16), jnp.float32),
            jax.ShapeDtypeStruct((W_MAX, 1), jnp.int32),
            jax.ShapeDtypeStruct((1, 1), jnp.int32),
        ),
    )(xf, Wg)


def _cast_body(w_ref, out_ref):
    out_ref[...] = w_ref[...].astype(jnp.bfloat16)


def _cast_bf16(arr):
    d0, d1, d2 = arr.shape
    return pl.pallas_call(
        _cast_body,
        grid=(d0,),
        in_specs=[pl.BlockSpec((1, d1, d2), lambda i: (i, 0, 0))],
        out_specs=pl.BlockSpec((1, d1, d2), lambda i: (i, 0, 0)),
        out_shape=jax.ShapeDtypeStruct((d0, d1, d2), jnp.bfloat16),
    )(arr)


def _sc_dispatch(xf, pos0, pos1):
    info = plsc.get_sparse_core_info()
    nw = info.num_cores * info.num_subcores
    t_per_w = S // nw
    chunk = 32
    n_chunks = t_per_w // chunk
    mesh = plsc.VectorSubcoreMesh(core_axis_name="c", subcore_axis_name="s")

    @functools.partial(
        pl.kernel, mesh=mesh,
        out_type=jax.ShapeDtypeStruct((R_PAD, DIM), jnp.float32),
        scratch_types=[
            pltpu.VMEM((chunk,), jnp.int32),
            pltpu.VMEM((chunk,), jnp.int32),
            pltpu.VMEM((chunk, DIM), jnp.float32),
            pltpu.SemaphoreType.DMA,
            pltpu.SemaphoreType.DMA,
        ],
    )
    def k(xf_hbm, p0_hbm, p1_hbm, out_hbm, p0_v, p1_v, rows_v, sem0, sem1):
        wid = lax.axis_index("s") * info.num_cores + lax.axis_index("c")
        for c in range(n_chunks):
            base = wid * t_per_w + c * chunk
            pltpu.sync_copy(p0_hbm.at[pl.ds(base, chunk)], p0_v)
            pltpu.sync_copy(p1_hbm.at[pl.ds(base, chunk)], p1_v)
            pltpu.sync_copy(xf_hbm.at[pl.ds(base, chunk)], rows_v)
            cp0 = pltpu.async_copy(rows_v, out_hbm.at[p0_v], sem0)
            cp1 = pltpu.async_copy(rows_v, out_hbm.at[p1_v], sem1)
            cp0.wait()
            cp1.wait()

    return k(xf, pos0, pos1)


def _sc_combine(rows, pos0, pos1, w0, w1):
    info = plsc.get_sparse_core_info()
    nw = info.num_cores * info.num_subcores
    t_per_w = S // nw
    chunk = 32
    n_chunks = t_per_w // chunk
    mesh = plsc.VectorSubcoreMesh(core_axis_name="c", subcore_axis_name="s")

    @functools.partial(
        pl.kernel, mesh=mesh,
        out_type=jax.ShapeDtypeStruct((S, DIM), jnp.float32),
        scratch_types=[
            pltpu.VMEM((chunk,), jnp.int32),
            pltpu.VMEM((chunk,), jnp.int32),
            pltpu.VMEM((chunk, 16), jnp.float32),
            pltpu.VMEM((chunk, 16), jnp.float32),
            pltpu.VMEM((chunk, DIM), jnp.float32),
            pltpu.VMEM((chunk, DIM), jnp.float32),
            pltpu.SemaphoreType.DMA,
            pltpu.SemaphoreType.DMA,
        ],
    )
    def k(rows_hbm, p0_hbm, p1_hbm, w0_hbm, w1_hbm, y_hbm,
          p0_v, p1_v, w0_v, w1_v, a_v, b_v, sem0, sem1):
        wid = lax.axis_index("s") * info.num_cores + lax.axis_index("c")
        for c in range(n_chunks):
            base = wid * t_per_w + c * chunk
            pltpu.sync_copy(p0_hbm.at[pl.ds(base, chunk)], p0_v)
            pltpu.sync_copy(p1_hbm.at[pl.ds(base, chunk)], p1_v)
            pltpu.sync_copy(w0_hbm.at[pl.ds(base, chunk)], w0_v)
            pltpu.sync_copy(w1_hbm.at[pl.ds(base, chunk)], w1_v)
            cp0 = pltpu.async_copy(rows_hbm.at[p0_v], a_v, sem0)
            cp1 = pltpu.async_copy(rows_hbm.at[p1_v], b_v, sem1)
            cp0.wait()
            cp1.wait()

            def body(t, _):
                wa = w0_v[t, :]
                wb = w1_v[t, :]

                def inner(j, _):
                    sl = pl.ds(j * 16, 16)
                    a_v[t, sl] = wa * a_v[t, sl] + wb * b_v[t, sl]
                    return 0

                lax.fori_loop(0, DIM // 16, inner, 0)
                return 0

            lax.fori_loop(0, chunk, body, 0)
            pltpu.sync_copy(a_v, y_hbm.at[pl.ds(base, chunk)])

    return k(rows, pos0, pos1, w0, w1)


HC = 1408
NHC = HIDDEN // HC


def _ffn_body(we_ref, used_ref, xs_ref, w1_ref, w3_ref, w2_ref, out_ref):
    i = pl.program_id(0)
    c = pl.program_id(1)

    @pl.when(i < used_ref[0])
    def _():
        x = xs_ref[...].astype(jnp.bfloat16)
        w1 = w1_ref[0].astype(jnp.bfloat16)
        w3 = w3_ref[0].astype(jnp.bfloat16)
        w2 = w2_ref[0].astype(jnp.bfloat16)
        h1 = lax.dot_general(x, w1, (((1,), (1,)), ((), ())),
                             preferred_element_type=jnp.float32)
        h3 = lax.dot_general(x, w3, (((1,), (1,)), ((), ())),
                             preferred_element_type=jnp.float32)
        h = (h1 * jax.nn.sigmoid(h1)) * h3
        y = lax.dot_general(h.astype(jnp.bfloat16), w2,
                            (((1,), (1,)), ((), ())),
                            preferred_element_type=jnp.float32)

        @pl.when(c == 0)
        def _():
            out_ref[...] = y

        @pl.when(c > 0)
        def _():
            out_ref[...] = out_ref[...] + y


def _ffn(we, used, xs, W1b, W3b, W2b):
    grid_spec = pltpu.PrefetchScalarGridSpec(
        num_scalar_prefetch=2,
        grid=(W_MAX, NHC),
        in_specs=[
            pl.BlockSpec((BLK, DIM), lambda i, c, we, u: (i, 0)),
            pl.BlockSpec((1, HC, DIM), lambda i, c, we, u: (we[i], c, 0)),
            pl.BlockSpec((1, HC, DIM), lambda i, c, we, u: (we[i], c, 0)),
            pl.BlockSpec((1, DIM, HC), lambda i, c, we, u: (we[i], 0, c)),
        ],
        out_specs=pl.BlockSpec((BLK, DIM), lambda i, c, we, u: (i, 0)),
    )
    return pl.pallas_call(
        _ffn_body,
        grid_spec=grid_spec,
        out_shape=jax.ShapeDtypeStruct((R_PAD, DIM), jnp.float32),
        compiler_params=pltpu.CompilerParams(
            dimension_semantics=("arbitrary", "arbitrary")),
    )(we, used, xs, W1b, W3b, W2b)


@jax.jit
def kernel(x, Wg, W1, W2, W3):
    orig_shape = x.shape
    xf = x.reshape(-1, DIM)

    pos0_2d, pos1_2d, w0, w1, we_2d, used_2d = _gate(xf, Wg)
    pos0 = pos0_2d.reshape(S)
    pos1 = pos1_2d.reshape(S)
    we = we_2d.reshape(W_MAX)
    used = used_2d.reshape(1)

    xs = _sc_dispatch(xf, pos0, pos1)

    rows = _ffn(we, used, xs, W1, W3, W2)

    y = _sc_combine(rows, pos0, pos1, w0, w1)
    return y.reshape(orig_shape)


# speedup vs baseline: 1.1820x; 1.1820x over previous
import functools

import jax
import jax.numpy as jnp
from jax import lax
from jax.experimental import pallas as pl
from jax.experimental.pallas import tpu as pltpu
from jax.experimental.pallas import tpu_sc as plsc

E = 8
TOP_K = 2
DIM = 1024
HIDDEN = 2816
S = 2048
R = S * TOP_K
BLK = 256
R_PAD = R + E * BLK
W_MAX = R_PAD // BLK


def _shift_down(x, k):
    z = jnp.zeros((k,) + x.shape[1:], x.dtype)
    return jnp.concatenate([z, x[:-k]], axis=0)


def _shift_right(x, k):
    z = jnp.zeros(x.shape[:1] + (k,), x.dtype)
    return jnp.concatenate([z, x[:, :-k]], axis=1)


def _gate_body(x_ref, wg_ref, pos0_ref, pos1_ref, w0_ref, w1_ref,
               we_ref, used_ref):
    x = x_ref[...]
    wg = wg_ref[...]
    s = lax.dot_general(x, wg, (((1,), (1,)), ((), ())),
                        preferred_element_type=jnp.float32)
    col = lax.broadcasted_iota(jnp.int32, s.shape, 1)
    m1 = jnp.max(s, axis=1, keepdims=True)
    i1 = jnp.min(jnp.where(s == m1, col, E), axis=1, keepdims=True)
    oh0 = (col == i1)
    s2 = jnp.where(oh0, -jnp.inf, s)
    m2 = jnp.max(s2, axis=1, keepdims=True)
    i2 = jnp.min(jnp.where(s2 == m2, col, E), axis=1, keepdims=True)
    oh1 = (col == i2)
    e2 = jnp.exp(m2 - m1)
    w0_ref[...] = jnp.broadcast_to(1.0 / (1.0 + e2), (S, 16))
    w1_ref[...] = jnp.broadcast_to(e2 / (1.0 + e2), (S, 16))

    a = oh0.astype(jnp.int32) + oh1.astype(jnp.int32)
    incl = a
    k = 1
    while k < S:
        incl = incl + _shift_down(incl, k)
        k *= 2
    cumbef = incl - a
    cnt = incl[S - 1:S, :]
    blocks_e = (cnt + (BLK - 1)) // BLK
    padc = blocks_e * BLK
    ip = padc
    k = 1
    while k < E:
        ip = ip + _shift_right(ip, k)
        k *= 2
    starts_pad = ip - padc
    base = starts_pad + cumbef
    pos0_ref[...] = jnp.sum(jnp.where(oh0, base, 0), axis=1, keepdims=True)
    pos1_ref[...] = jnp.sum(jnp.where(oh1, base, 0), axis=1, keepdims=True)
    used_ref[...] = jnp.sum(blocks_e, axis=1, keepdims=True)
    start_blk = starts_pad // BLK
    blk = lax.broadcasted_iota(jnp.int32, (W_MAX, E), 0)
    ge = (blk >= jnp.broadcast_to(start_blk, (W_MAX, E))).astype(jnp.int32)
    we_ref[...] = jnp.sum(ge, axis=1, keepdims=True) - 1


def _gate(xf, Wg):
    return pl.pallas_call(
        _gate_body,
        out_shape=(
            jax.ShapeDtypeStruct((S, 1), jnp.int32),
            jax.ShapeDtypeStruct((S, 1), jnp.int32),
            jax.ShapeDtypeStruct((S, 16), jnp.float32),
            jax.ShapeDtypeStruct((S, 16), jnp.float32),
            jax.ShapeDtypeStruct((W_MAX, 1), jnp.int32),
            jax.ShapeDtypeStruct((1, 1), jnp.int32),
        ),
    )(xf, Wg)


def _sc_dispatch(xf, pos0, pos1):
    info = plsc.get_sparse_core_info()
    nw = info.num_cores * info.num_subcores
    t_per_w = S // nw
    chunk = 32
    n_chunks = t_per_w // chunk
    mesh = plsc.VectorSubcoreMesh(core_axis_name="c", subcore_axis_name="s")

    @functools.partial(
        pl.kernel, mesh=mesh,
        out_type=jax.ShapeDtypeStruct((R_PAD, DIM), jnp.float32),
        scratch_types=[
            pltpu.VMEM((chunk,), jnp.int32),
            pltpu.VMEM((chunk,), jnp.int32),
            pltpu.VMEM((chunk, DIM), jnp.float32),
            pltpu.SemaphoreType.DMA,
            pltpu.SemaphoreType.DMA,
        ],
    )
    def k(xf_hbm, p0_hbm, p1_hbm, out_hbm, p0_v, p1_v, rows_v, sem0, sem1):
        wid = lax.axis_index("s") * info.num_cores + lax.axis_index("c")
        for c in range(n_chunks):
            base = wid * t_per_w + c * chunk
            pltpu.sync_copy(p0_hbm.at[pl.ds(base, chunk)], p0_v)
            pltpu.sync_copy(p1_hbm.at[pl.ds(base, chunk)], p1_v)
            pltpu.sync_copy(xf_hbm.at[pl.ds(base, chunk)], rows_v)
            cp0 = pltpu.async_copy(rows_v, out_hbm.at[p0_v], sem0)
            cp1 = pltpu.async_copy(rows_v, out_hbm.at[p1_v], sem1)
            cp0.wait()
            cp1.wait()

    return k(xf, pos0, pos1)


def _sc_combine(rows, pos0, pos1, w0, w1):
    info = plsc.get_sparse_core_info()
    nw = info.num_cores * info.num_subcores
    t_per_w = S // nw
    chunk = 32
    n_chunks = t_per_w // chunk
    mesh = plsc.VectorSubcoreMesh(core_axis_name="c", subcore_axis_name="s")

    @functools.partial(
        pl.kernel, mesh=mesh,
        out_type=jax.ShapeDtypeStruct((S, DIM), jnp.float32),
        scratch_types=[
            pltpu.VMEM((chunk,), jnp.int32),
            pltpu.VMEM((chunk,), jnp.int32),
            pltpu.VMEM((chunk, 16), jnp.float32),
            pltpu.VMEM((chunk, 16), jnp.float32),
            pltpu.VMEM((chunk, DIM), jnp.float32),
            pltpu.VMEM((chunk, DIM), jnp.float32),
            pltpu.SemaphoreType.DMA,
            pltpu.SemaphoreType.DMA,
        ],
    )
    def k(rows_hbm, p0_hbm, p1_hbm, w0_hbm, w1_hbm, y_hbm,
          p0_v, p1_v, w0_v, w1_v, a_v, b_v, sem0, sem1):
        wid = lax.axis_index("s") * info.num_cores + lax.axis_index("c")
        for c in range(n_chunks):
            base = wid * t_per_w + c * chunk
            pltpu.sync_copy(p0_hbm.at[pl.ds(base, chunk)], p0_v)
            pltpu.sync_copy(p1_hbm.at[pl.ds(base, chunk)], p1_v)
            pltpu.sync_copy(w0_hbm.at[pl.ds(base, chunk)], w0_v)
            pltpu.sync_copy(w1_hbm.at[pl.ds(base, chunk)], w1_v)
            cp0 = pltpu.async_copy(rows_hbm.at[p0_v], a_v, sem0)
            cp1 = pltpu.async_copy(rows_hbm.at[p1_v], b_v, sem1)
            cp0.wait()
            cp1.wait()

            def body(t, _):
                wa = w0_v[t, :]
                wb = w1_v[t, :]

                def inner(j, _):
                    sl = pl.ds(j * 16, 16)
                    a_v[t, sl] = wa * a_v[t, sl] + wb * b_v[t, sl]
                    return 0

                lax.fori_loop(0, DIM // 16, inner, 0)
                return 0

            lax.fori_loop(0, chunk, body, 0)
            pltpu.sync_copy(a_v, y_hbm.at[pl.ds(base, chunk)])

    return k(rows, pos0, pos1, w0, w1)


HC = 1408
NHC = HIDDEN // HC


def _make_ffn_half_body(first):
    def body(we_ref, used_ref, xs_ref, w1_ref, w3_ref, w2_ref, *rest):
        if first:
            out_ref, w1s, w3s, w2s = rest
        else:
            yin_ref, out_ref, w1s, w3s, w2s = rest
        i = pl.program_id(0)

        @pl.when(i < used_ref[0])
        def _():
            prev = jnp.where(i > 0, we_ref[jnp.maximum(i - 1, 0)], -1)

            @pl.when(we_ref[i] != prev)
            def _():
                w1s[...] = w1_ref[0].astype(jnp.bfloat16)
                w3s[...] = w3_ref[0].astype(jnp.bfloat16)
                w2s[...] = w2_ref[0].astype(jnp.bfloat16)

            x = xs_ref[...].astype(jnp.bfloat16)
            h1 = lax.dot_general(x, w1s[...], (((1,), (1,)), ((), ())),
                                 preferred_element_type=jnp.float32)
            h3 = lax.dot_general(x, w3s[...], (((1,), (1,)), ((), ())),
                                 preferred_element_type=jnp.float32)
            h = (h1 * jax.nn.sigmoid(h1)) * h3
            y = lax.dot_general(h.astype(jnp.bfloat16), w2s[...],
                                (((1,), (1,)), ((), ())),
                                preferred_element_type=jnp.float32)
            if first:
                out_ref[...] = y
            else:
                out_ref[...] = y + yin_ref[...]

    return body


def _ffn_half(we, used, xs, W1, W3, W2, c, yin):
    first = yin is None
    in_specs = [
        pl.BlockSpec((BLK, DIM), lambda i, we, u: (i, 0)),
        pl.BlockSpec((1, HC, DIM), lambda i, we, u: (we[i], c, 0)),
        pl.BlockSpec((1, HC, DIM), lambda i, we, u: (we[i], c, 0)),
        pl.BlockSpec((1, DIM, HC), lambda i, we, u: (we[i], 0, c)),
    ]
    args = [we, used, xs, W1, W3, W2]
    if not first:
        in_specs.append(pl.BlockSpec((BLK, DIM), lambda i, we, u: (i, 0)))
        args.append(yin)
    grid_spec = pltpu.PrefetchScalarGridSpec(
        num_scalar_prefetch=2,
        grid=(W_MAX,),
        in_specs=in_specs,
        out_specs=pl.BlockSpec((BLK, DIM), lambda i, we, u: (i, 0)),
        scratch_shapes=[
            pltpu.VMEM((HC, DIM), jnp.bfloat16),
            pltpu.VMEM((HC, DIM), jnp.bfloat16),
            pltpu.VMEM((DIM, HC), jnp.bfloat16),
        ],
    )
    return pl.pallas_call(
        _make_ffn_half_body(first),
        grid_spec=grid_spec,
        out_shape=jax.ShapeDtypeStruct((R_PAD, DIM), jnp.float32),
        compiler_params=pltpu.CompilerParams(
            dimension_semantics=("arbitrary",)),
    )(*args)


def _ffn(we, used, xs, W1, W3, W2):
    y = _ffn_half(we, used, xs, W1, W3, W2, 0, None)
    y = _ffn_half(we, used, xs, W1, W3, W2, 1, y)
    return y


@jax.jit
def kernel(x, Wg, W1, W2, W3):
    orig_shape = x.shape
    xf = x.reshape(-1, DIM)

    pos0_2d, pos1_2d, w0, w1, we_2d, used_2d = _gate(xf, Wg)
    pos0 = pos0_2d.reshape(S)
    pos1 = pos1_2d.reshape(S)
    we = we_2d.reshape(W_MAX)
    used = used_2d.reshape(1)

    xs = _sc_dispatch(xf, pos0, pos1)

    rows = _ffn(we, used, xs, W1, W3, W2)

    y = _sc_combine(rows, pos0, pos1, w0, w1)
    return y.reshape(orig_shape)


# speedup vs baseline: 1.2968x; 1.0972x over previous
import functools

import jax
import jax.numpy as jnp
from jax import lax
from jax.experimental import pallas as pl
from jax.experimental.pallas import tpu as pltpu
from jax.experimental.pallas import tpu_sc as plsc

E = 8
TOP_K = 2
DIM = 1024
HIDDEN = 2816
S = 2048
R = S * TOP_K
BLK = 512
R_PAD = R + E * BLK
W_MAX = R_PAD // BLK


def _shift_down(x, k):
    z = jnp.zeros((k,) + x.shape[1:], x.dtype)
    return jnp.concatenate([z, x[:-k]], axis=0)


def _shift_right(x, k):
    z = jnp.zeros(x.shape[:1] + (k,), x.dtype)
    return jnp.concatenate([z, x[:, :-k]], axis=1)


def _gate_body(x_ref, wg_ref, pos0_ref, pos1_ref, w0_ref, w1_ref,
               we_ref, used_ref):
    x = x_ref[...]
    wg = wg_ref[...]
    s = lax.dot_general(x, wg, (((1,), (1,)), ((), ())),
                        preferred_element_type=jnp.float32)
    col = lax.broadcasted_iota(jnp.int32, s.shape, 1)
    m1 = jnp.max(s, axis=1, keepdims=True)
    i1 = jnp.min(jnp.where(s == m1, col, E), axis=1, keepdims=True)
    oh0 = (col == i1)
    s2 = jnp.where(oh0, -jnp.inf, s)
    m2 = jnp.max(s2, axis=1, keepdims=True)
    i2 = jnp.min(jnp.where(s2 == m2, col, E), axis=1, keepdims=True)
    oh1 = (col == i2)
    e2 = jnp.exp(m2 - m1)
    w0_ref[...] = jnp.broadcast_to(1.0 / (1.0 + e2), (S, 16))
    w1_ref[...] = jnp.broadcast_to(e2 / (1.0 + e2), (S, 16))

    a = oh0.astype(jnp.int32) + oh1.astype(jnp.int32)
    incl = a
    k = 1
    while k < S:
        incl = incl + _shift_down(incl, k)
        k *= 2
    cumbef = incl - a
    cnt = incl[S - 1:S, :]
    blocks_e = (cnt + (BLK - 1)) // BLK
    padc = blocks_e * BLK
    ip = padc
    k = 1
    while k < E:
        ip = ip + _shift_right(ip, k)
        k *= 2
    starts_pad = ip - padc
    base = starts_pad + cumbef
    pos0_ref[...] = jnp.sum(jnp.where(oh0, base, 0), axis=1, keepdims=True)
    pos1_ref[...] = jnp.sum(jnp.where(oh1, base, 0), axis=1, keepdims=True)
    used_ref[...] = jnp.sum(blocks_e, axis=1, keepdims=True)
    start_blk = starts_pad // BLK
    blk = lax.broadcasted_iota(jnp.int32, (W_MAX, E), 0)
    ge = (blk >= jnp.broadcast_to(start_blk, (W_MAX, E))).astype(jnp.int32)
    we_ref[...] = jnp.sum(ge, axis=1, keepdims=True) - 1


def _gate(xf, Wg):
    return pl.pallas_call(
        _gate_body,
        out_shape=(
            jax.ShapeDtypeStruct((S, 1), jnp.int32),
            jax.ShapeDtypeStruct((S, 1), jnp.int32),
            jax.ShapeDtypeStruct((S, 16), jnp.float32),
            jax.ShapeDtypeStruct((S, 16), jnp.float32),
            jax.ShapeDtypeStruct((W_MAX, 1), jnp.int32),
            jax.ShapeDtypeStruct((1, 1), jnp.int32),
        ),
    )(xf, Wg)


def _sc_dispatch(xf, pos0, pos1):
    info = plsc.get_sparse_core_info()
    nw = info.num_cores * info.num_subcores
    t_per_w = S // nw
    chunk = 32
    n_chunks = t_per_w // chunk
    mesh = plsc.VectorSubcoreMesh(core_axis_name="c", subcore_axis_name="s")

    @functools.partial(
        pl.kernel, mesh=mesh,
        out_type=jax.ShapeDtypeStruct((R_PAD, DIM), jnp.float32),
        scratch_types=[
            pltpu.VMEM((chunk,), jnp.int32),
            pltpu.VMEM((chunk,), jnp.int32),
            pltpu.VMEM((chunk, DIM), jnp.float32),
            pltpu.SemaphoreType.DMA,
            pltpu.SemaphoreType.DMA,
        ],
    )
    def k(xf_hbm, p0_hbm, p1_hbm, out_hbm, p0_v, p1_v, rows_v, sem0, sem1):
        wid = lax.axis_index("s") * info.num_cores + lax.axis_index("c")
        for c in range(n_chunks):
            base = wid * t_per_w + c * chunk
            pltpu.sync_copy(p0_hbm.at[pl.ds(base, chunk)], p0_v)
            pltpu.sync_copy(p1_hbm.at[pl.ds(base, chunk)], p1_v)
            pltpu.sync_copy(xf_hbm.at[pl.ds(base, chunk)], rows_v)
            cp0 = pltpu.async_copy(rows_v, out_hbm.at[p0_v], sem0)
            cp1 = pltpu.async_copy(rows_v, out_hbm.at[p1_v], sem1)
            cp0.wait()
            cp1.wait()

    return k(xf, pos0, pos1)


def _sc_combine(rows, pos0, pos1, w0, w1):
    info = plsc.get_sparse_core_info()
    nw = info.num_cores * info.num_subcores
    t_per_w = S // nw
    chunk = 32
    n_chunks = t_per_w // chunk
    mesh = plsc.VectorSubcoreMesh(core_axis_name="c", subcore_axis_name="s")

    @functools.partial(
        pl.kernel, mesh=mesh,
        out_type=jax.ShapeDtypeStruct((S, DIM), jnp.float32),
        scratch_types=[
            pltpu.VMEM((chunk,), jnp.int32),
            pltpu.VMEM((chunk,), jnp.int32),
            pltpu.VMEM((chunk, 16), jnp.float32),
            pltpu.VMEM((chunk, 16), jnp.float32),
            pltpu.VMEM((chunk, DIM), jnp.float32),
            pltpu.VMEM((chunk, DIM), jnp.float32),
            pltpu.SemaphoreType.DMA,
            pltpu.SemaphoreType.DMA,
        ],
    )
    def k(rows_hbm, p0_hbm, p1_hbm, w0_hbm, w1_hbm, y_hbm,
          p0_v, p1_v, w0_v, w1_v, a_v, b_v, sem0, sem1):
        wid = lax.axis_index("s") * info.num_cores + lax.axis_index("c")
        for c in range(n_chunks):
            base = wid * t_per_w + c * chunk
            pltpu.sync_copy(p0_hbm.at[pl.ds(base, chunk)], p0_v)
            pltpu.sync_copy(p1_hbm.at[pl.ds(base, chunk)], p1_v)
            pltpu.sync_copy(w0_hbm.at[pl.ds(base, chunk)], w0_v)
            pltpu.sync_copy(w1_hbm.at[pl.ds(base, chunk)], w1_v)
            cp0 = pltpu.async_copy(rows_hbm.at[p0_v], a_v, sem0)
            cp1 = pltpu.async_copy(rows_hbm.at[p1_v], b_v, sem1)
            cp0.wait()
            cp1.wait()

            def body(t, _):
                wa = w0_v[t, :]
                wb = w1_v[t, :]

                def inner(j, _):
                    sl = pl.ds(j * 16, 16)
                    a_v[t, sl] = wa * a_v[t, sl] + wb * b_v[t, sl]
                    return 0

                lax.fori_loop(0, DIM // 16, inner, 0)
                return 0

            lax.fori_loop(0, chunk, body, 0)
            pltpu.sync_copy(a_v, y_hbm.at[pl.ds(base, chunk)])

    return k(rows, pos0, pos1, w0, w1)


HC = 1408
NHC = HIDDEN // HC


def _make_ffn_half_body(first):
    def body(we_ref, used_ref, xs_ref, w1_ref, w3_ref, w2_ref, *rest):
        if first:
            out_ref, w1s, w3s, w2s = rest
        else:
            yin_ref, out_ref, w1s, w3s, w2s = rest
        i = pl.program_id(0)

        @pl.when(i < used_ref[0])
        def _():
            prev = jnp.where(i > 0, we_ref[jnp.maximum(i - 1, 0)], -1)

            @pl.when(we_ref[i] != prev)
            def _():
                w1s[...] = w1_ref[0].astype(jnp.bfloat16)
                w3s[...] = w3_ref[0].astype(jnp.bfloat16)
                w2s[...] = w2_ref[0].astype(jnp.bfloat16)

            x = xs_ref[...].astype(jnp.bfloat16)
            h1 = lax.dot_general(x, w1s[...], (((1,), (1,)), ((), ())),
                                 preferred_element_type=jnp.float32)
            h3 = lax.dot_general(x, w3s[...], (((1,), (1,)), ((), ())),
                                 preferred_element_type=jnp.float32)
            h = (h1 * jax.nn.sigmoid(h1)) * h3
            y = lax.dot_general(h.astype(jnp.bfloat16), w2s[...],
                                (((1,), (1,)), ((), ())),
                                preferred_element_type=jnp.float32)
            if first:
                out_ref[...] = y
            else:
                out_ref[...] = y + yin_ref[...]

    return body


def _ffn_half(we, used, xs, W1, W3, W2, c, yin):
    first = yin is None
    in_specs = [
        pl.BlockSpec((BLK, DIM), lambda i, we, u: (i, 0)),
        pl.BlockSpec((1, HC, DIM), lambda i, we, u: (we[i], c, 0)),
        pl.BlockSpec((1, HC, DIM), lambda i, we, u: (we[i], c, 0)),
        pl.BlockSpec((1, DIM, HC), lambda i, we, u: (we[i], 0, c)),
    ]
    args = [we, used, xs, W1, W3, W2]
    if not first:
        in_specs.append(pl.BlockSpec((BLK, DIM), lambda i, we, u: (i, 0)))
        args.append(yin)
    grid_spec = pltpu.PrefetchScalarGridSpec(
        num_scalar_prefetch=2,
        grid=(W_MAX,),
        in_specs=in_specs,
        out_specs=pl.BlockSpec((BLK, DIM), lambda i, we, u: (i, 0)),
        scratch_shapes=[
            pltpu.VMEM((HC, DIM), jnp.bfloat16),
            pltpu.VMEM((HC, DIM), jnp.bfloat16),
            pltpu.VMEM((DIM, HC), jnp.bfloat16),
        ],
    )
    return pl.pallas_call(
        _make_ffn_half_body(first),
        grid_spec=grid_spec,
        out_shape=jax.ShapeDtypeStruct((R_PAD, DIM), jnp.float32),
        compiler_params=pltpu.CompilerParams(
            dimension_semantics=("arbitrary",)),
    )(*args)


def _ffn(we, used, xs, W1, W3, W2):
    y = _ffn_half(we, used, xs, W1, W3, W2, 0, None)
    y = _ffn_half(we, used, xs, W1, W3, W2, 1, y)
    return y


@jax.jit
def kernel(x, Wg, W1, W2, W3):
    orig_shape = x.shape
    xf = x.reshape(-1, DIM)

    pos0_2d, pos1_2d, w0, w1, we_2d, used_2d = _gate(xf, Wg)
    pos0 = pos0_2d.reshape(S)
    pos1 = pos1_2d.reshape(S)
    we = we_2d.reshape(W_MAX)
    used = used_2d.reshape(1)

    xs = _sc_dispatch(xf, pos0, pos1)

    rows = _ffn(we, used, xs, W1, W3, W2)

    y = _sc_combine(rows, pos0, pos1, w0, w1)
    return y.reshape(orig_shape)
